# baseline (speedup 1.0000x reference)
import functools

import jax
import jax.numpy as jnp
from jax import lax
from jax.experimental import pallas as pl
from jax.experimental.pallas import tpu as pltpu
from jax.experimental.pallas import tpu_sc as plsc

N = 10000
D_IN = 256
H1 = 256
H2 = 256
C = 128

E = 320000
CHUNK = 128
NTILES = 32
CH_PER_TILE = 8 * (-(-E // (CHUNK * NTILES * 8)))
NCHUNKS = CH_PER_TILE * NTILES
EPAD = NCHUNKS * CHUNK
PIECE = 32
NP_FAST = 3
NP_SLOW = 2
FAST_CHUNKS = 16 * NP_FAST * PIECE
assert 16 * (NP_FAST + NP_SLOW) * PIECE == NCHUNKS
NPAD = 10112
ROWS_PER_TILE = NPAD // 16
DEG_W = C

BLK = 1000
NBLK = N // BLK

_mesh = plsc.VectorSubcoreMesh(core_axis_name="c", subcore_axis_name="s")


@functools.partial(
    pl.kernel,
    mesh=_mesh,
    out_type=jax.ShapeDtypeStruct((2, NPAD, DEG_W), jnp.float32),
    scratch_types=[
        pltpu.VMEM((CH_PER_TILE, CHUNK), jnp.int32),
        pltpu.VMEM((CHUNK, DEG_W), jnp.float32),
        pltpu.VMEM_SHARED((NPAD, DEG_W), jnp.float32),
    ],
)
def _sc_degree(dst_hbm, ones_hbm, zeros_hbm, out_hbm, dst_buf, ones_buf, acc):
    c = lax.axis_index("c")
    s = lax.axis_index("s")
    rbase = s * ROWS_PER_TILE
    pltpu.sync_copy(zeros_hbm.at[pl.ds(rbase, ROWS_PER_TILE)],
                    acc.at[pl.ds(rbase, ROWS_PER_TILE)])
    pltpu.sync_copy(ones_hbm, ones_buf)
    cbase = (c * 16 + s) * CH_PER_TILE
    pltpu.sync_copy(dst_hbm.at[pl.ds(cbase, CH_PER_TILE)], dst_buf)
    plsc.subcore_barrier()

    def body(j, carry):
        pltpu.sync_copy(ones_buf, acc.at[dst_buf.at[j]], add=True)
        return carry

    lax.fori_loop(0, CH_PER_TILE, body, 0)
    plsc.subcore_barrier()
    pltpu.sync_copy(acc.at[pl.ds(rbase, ROWS_PER_TILE)],
                    out_hbm.at[c].at[pl.ds(rbase, ROWS_PER_TILE)])


@functools.partial(
    pl.kernel,
    mesh=_mesh,
    out_type=jax.ShapeDtypeStruct((2, NPAD, C), jnp.float32),
    scratch_types=[
        pltpu.VMEM((PIECE, CHUNK), jnp.int32),
        pltpu.VMEM((PIECE, CHUNK), jnp.int32),
    ]
    + [pltpu.VMEM((CHUNK, C), jnp.float32) for _ in range(2)]
    + [pltpu.VMEM_SHARED((NPAD, C), jnp.float32)]
    + [pltpu.SemaphoreType.DMA for _ in range(2)],
)
def _sc_edge_scatter(hs_hbm, src_hbm, dst_hbm, zeros_hbm, out_hbm,
                     src_buf, dst_buf, rb0, rb1, acc, g0, g1):
    NBUF = 2
    NGRP = PIECE // NBUF
    rbufs = (rb0, rb1)
    gsems = (g0, g1)
    c = lax.axis_index("c")
    s = lax.axis_index("s")
    rbase = s * ROWS_PER_TILE
    pltpu.sync_copy(zeros_hbm.at[pl.ds(rbase, ROWS_PER_TILE)],
                    acc.at[pl.ds(rbase, ROWS_PER_TILE)])
    plsc.subcore_barrier()
    npieces = jnp.where(c == 0, NP_FAST, NP_SLOW)
    tbase = jnp.where(c == 0, s * NP_FAST * PIECE,
                      FAST_CHUNKS + s * NP_SLOW * PIECE)

    def piece(p, carry):
        pbase = tbase + p * PIECE
        pltpu.sync_copy(src_hbm.at[pl.ds(pbase, PIECE)], src_buf)
        pltpu.sync_copy(dst_hbm.at[pl.ds(pbase, PIECE)], dst_buf)
        for b in range(NBUF):
            pltpu.async_copy(hs_hbm.at[src_buf.at[b]], rbufs[b], gsems[b])

        def group(g, carry2):
            for b in range(NBUF):
                j = g * NBUF + b
                pltpu.make_async_copy(hs_hbm.at[src_buf.at[j]],
                                      rbufs[b], gsems[b]).wait()
                pltpu.sync_copy(rbufs[b], acc.at[dst_buf.at[j]], add=True)
                pltpu.async_copy(
                    hs_hbm.at[src_buf.at[lax.rem(j + NBUF, PIECE)]],
                    rbufs[b], gsems[b])
            return carry2

        lax.fori_loop(0, NGRP, group, 0)
        for b in range(NBUF):
            pltpu.make_async_copy(hs_hbm.at[src_buf.at[b]],
                                  rbufs[b], gsems[b]).wait()
        return carry

    lax.fori_loop(0, npieces, piece, 0)
    plsc.subcore_barrier()
    pltpu.sync_copy(acc.at[pl.ds(rbase, ROWS_PER_TILE)],
                    out_hbm.at[c].at[pl.ds(rbase, ROWS_PER_TILE)])


def _dis_from(degp_ref):
    deg = degp_ref[0, :, :1] + degp_ref[1, :, :1] + 1.0
    return lax.rsqrt(deg)


def _leaky(v):
    return jnp.where(v > 0, v, 0.01 * v)


def _tc_mlp_body(x_ref, w1_ref, b1_ref, w2_ref, b2_ref, wc1_ref, bc1_ref,
                 degp_ref, hs_ref):
    h = _leaky(jnp.dot(x_ref[...], w1_ref[...],
                       preferred_element_type=jnp.float32) + b1_ref[...])
    h = _leaky(jnp.dot(h, w2_ref[...],
                       preferred_element_type=jnp.float32) + b2_ref[...])
    ht = jnp.dot(h, wc1_ref[...],
                 preferred_element_type=jnp.float32) + bc1_ref[...]
    hs_ref[...] = ht * _dis_from(degp_ref)


def _tc_mid_body(p_ref, hs1_ref, degp_ref, wc2_ref, bc2_ref, hs2_ref):
    dis = _dis_from(degp_ref)
    c1 = _leaky(dis * (p_ref[0] + p_ref[1] + hs1_ref[...]))
    nrm = jnp.sqrt(jnp.sum(c1 * c1, axis=1, keepdims=True))
    hn = c1 / nrm
    ht2 = jnp.dot(hn, wc2_ref[...],
                  preferred_element_type=jnp.float32) + bc2_ref[...]
    hs2_ref[...] = ht2 * dis


def _tc_final_body(q_ref, hs2_ref, degp_ref, out_ref):
    dis = _dis_from(degp_ref)
    out_ref[...] = _leaky(dis * (q_ref[0] + q_ref[1] + hs2_ref[...]))


def _full(shape):
    return pl.BlockSpec(shape, lambda i: (0,) * len(shape))


_row_spec = lambda w: pl.BlockSpec((BLK, w), lambda i: (i, 0))
_part_spec = lambda w: pl.BlockSpec((2, BLK, w), lambda i: (0, i, 0))

_tc_mlp = pl.pallas_call(
    _tc_mlp_body,
    grid=(NBLK,),
    in_specs=[
        _row_spec(D_IN), _full((D_IN, H1)), _full((1, H1)),
        _full((H1, H2)), _full((1, H2)), _full((H2, C)), _full((1, C)),
        _part_spec(DEG_W),
    ],
    out_specs=_row_spec(C),
    out_shape=jax.ShapeDtypeStruct((N, C), jnp.float32),
)

_tc_mid = pl.pallas_call(
    _tc_mid_body,
    grid=(NBLK,),
    in_specs=[
        _part_spec(C), _row_spec(C), _part_spec(DEG_W),
        _full((C, C)), _full((1, C)),
    ],
    out_specs=_row_spec(C),
    out_shape=jax.ShapeDtypeStruct((N, C), jnp.float32),
)

_tc_final = pl.pallas_call(
    _tc_final_body,
    grid=(NBLK,),
    in_specs=[_part_spec(C), _row_spec(C), _part_spec(DEG_W)],
    out_specs=_row_spec(C),
    out_shape=jax.ShapeDtypeStruct((N, C), jnp.float32),
)


def kernel(x, edge_index, previous_embeddings_0, previous_embeddings_1,
           W1, b1, W2, b2, Wc1, bc1, Wc2, bc2):
    del previous_embeddings_0, previous_embeddings_1
    src = edge_index[0]
    dst = edge_index[1]
    src_p = jnp.concatenate(
        [src, jnp.zeros((EPAD - E,), jnp.int32)]).reshape(NCHUNKS, CHUNK)
    dst_p = jnp.concatenate(
        [dst, jnp.full((EPAD - E,), N, jnp.int32)]).reshape(NCHUNKS, CHUNK)
    ones_deg = jnp.ones((CHUNK, DEG_W), jnp.float32)
    zeros_c = jnp.zeros((NPAD, C), jnp.float32)

    degp = _sc_degree(dst_p, ones_deg, zeros_c)
    hs1 = _tc_mlp(x, W1, b1.reshape(1, -1), W2, b2.reshape(1, -1),
                  Wc1, bc1.reshape(1, -1), degp)
    p1 = _sc_edge_scatter(hs1, src_p, dst_p, zeros_c)
    hs2 = _tc_mid(p1, hs1, degp, Wc2, bc2.reshape(1, -1))
    p2 = _sc_edge_scatter(hs2, src_p, dst_p, zeros_c)
    return _tc_final(p2, hs2, degp)


# speedup vs baseline: 2.8397x; 2.8397x over previous
import functools

import jax
import jax.numpy as jnp
from jax import lax
from jax.experimental import pallas as pl
from jax.experimental.pallas import tpu as pltpu
from jax.experimental.pallas import tpu_sc as plsc

N = 10000
D_IN = 256
H1 = 256
H2 = 256
C = 128

E = 320000
CHUNK = 128
NTILES = 32
CH_PER_TILE = 8 * (-(-E // (CHUNK * NTILES * 8)))
NCHUNKS = CH_PER_TILE * NTILES
EPAD = NCHUNKS * CHUNK
PIECE = 40
NP_FAST = 2
NP_SLOW = 2
FAST_CHUNKS = 16 * NP_FAST * PIECE
assert 16 * (NP_FAST + NP_SLOW) * PIECE == NCHUNKS
NPAD = 10112
ROWS_PER_TILE = NPAD // 16
DEG_W = C

BLK = 1000
NBLK = N // BLK

_mesh = plsc.VectorSubcoreMesh(core_axis_name="c", subcore_axis_name="s")


@functools.partial(
    pl.kernel,
    mesh=_mesh,
    out_type=jax.ShapeDtypeStruct((2, NPAD, DEG_W), jnp.float32),
    scratch_types=[
        pltpu.VMEM((CH_PER_TILE, CHUNK), jnp.int32),
        pltpu.VMEM((CHUNK, DEG_W), jnp.float32),
        pltpu.VMEM_SHARED((NPAD, DEG_W), jnp.float32),
    ],
)
def _sc_degree(dst_hbm, ones_hbm, zeros_hbm, out_hbm, dst_buf, ones_buf, acc):
    c = lax.axis_index("c")
    s = lax.axis_index("s")
    rbase = s * ROWS_PER_TILE
    pltpu.sync_copy(zeros_hbm.at[pl.ds(rbase, ROWS_PER_TILE)],
                    acc.at[pl.ds(rbase, ROWS_PER_TILE)])
    pltpu.sync_copy(ones_hbm, ones_buf)
    cbase = (c * 16 + s) * CH_PER_TILE
    pltpu.sync_copy(dst_hbm.at[pl.ds(cbase, CH_PER_TILE)], dst_buf)
    plsc.subcore_barrier()

    def body(j, carry):
        pltpu.sync_copy(ones_buf, acc.at[dst_buf.at[j]], add=True)
        return carry

    lax.fori_loop(0, CH_PER_TILE, body, 0)
    plsc.subcore_barrier()
    pltpu.sync_copy(acc.at[pl.ds(rbase, ROWS_PER_TILE)],
                    out_hbm.at[c].at[pl.ds(rbase, ROWS_PER_TILE)])


@functools.partial(
    pl.kernel,
    mesh=_mesh,
    out_type=jax.ShapeDtypeStruct((2, NPAD, C), jnp.float32),
    scratch_types=[
        pltpu.VMEM((PIECE, CHUNK), jnp.int32),
        pltpu.VMEM((PIECE, CHUNK), jnp.int32),
    ]
    + [pltpu.VMEM((CHUNK, C), jnp.float32) for _ in range(2)]
    + [pltpu.VMEM_SHARED((NPAD, C), jnp.float32)]
    + [pltpu.SemaphoreType.DMA for _ in range(2)],
)
def _sc_edge_scatter(hs_hbm, src_hbm, dst_hbm, zeros_hbm, out_hbm,
                     src_buf, dst_buf, rb0, rb1, acc, g0, g1):
    NBUF = 2
    NGRP = PIECE // NBUF
    rbufs = (rb0, rb1)
    gsems = (g0, g1)
    c = lax.axis_index("c")
    s = lax.axis_index("s")
    rbase = s * ROWS_PER_TILE
    pltpu.sync_copy(zeros_hbm.at[pl.ds(rbase, ROWS_PER_TILE)],
                    acc.at[pl.ds(rbase, ROWS_PER_TILE)])
    plsc.subcore_barrier()
    npieces = jnp.where(c == 0, NP_FAST, NP_SLOW)
    tbase = jnp.where(c == 0, s * NP_FAST * PIECE,
                      FAST_CHUNKS + s * NP_SLOW * PIECE)

    def piece(p, carry):
        pbase = tbase + p * PIECE
        pltpu.sync_copy(src_hbm.at[pl.ds(pbase, PIECE)], src_buf)
        pltpu.sync_copy(dst_hbm.at[pl.ds(pbase, PIECE)], dst_buf)
        for b in range(NBUF):
            pltpu.async_copy(hs_hbm.at[src_buf.at[b]], rbufs[b], gsems[b])

        def group(g, carry2):
            for b in range(NBUF):
                j = g * NBUF + b
                pltpu.make_async_copy(hs_hbm.at[src_buf.at[j]],
                                      rbufs[b], gsems[b]).wait()
                pltpu.sync_copy(rbufs[b], acc.at[dst_buf.at[j]], add=True)
                pltpu.async_copy(
                    hs_hbm.at[src_buf.at[lax.rem(j + NBUF, PIECE)]],
                    rbufs[b], gsems[b])
            return carry2

        lax.fori_loop(0, NGRP, group, 0)
        for b in range(NBUF):
            pltpu.make_async_copy(hs_hbm.at[src_buf.at[b]],
                                  rbufs[b], gsems[b]).wait()
        return carry

    lax.fori_loop(0, npieces, piece, 0)
    plsc.subcore_barrier()
    pltpu.sync_copy(acc.at[pl.ds(rbase, ROWS_PER_TILE)],
                    out_hbm.at[c].at[pl.ds(rbase, ROWS_PER_TILE)])


def _dis_from(degp_ref):
    deg = degp_ref[0, :, :1] + degp_ref[1, :, :1] + 1.0
    return lax.rsqrt(deg)


def _leaky(v):
    return jnp.where(v > 0, v, 0.01 * v)


def _tc_mlp_body(x_ref, w1_ref, b1_ref, w2_ref, b2_ref, wc1_ref, bc1_ref,
                 degp_ref, hs_ref):
    h = _leaky(jnp.dot(x_ref[...], w1_ref[...],
                       preferred_element_type=jnp.float32) + b1_ref[...])
    h = _leaky(jnp.dot(h, w2_ref[...],
                       preferred_element_type=jnp.float32) + b2_ref[...])
    ht = jnp.dot(h, wc1_ref[...],
                 preferred_element_type=jnp.float32) + bc1_ref[...]
    hs_ref[...] = ht * _dis_from(degp_ref)


def _tc_mid_body(p_ref, hs1_ref, degp_ref, wc2_ref, bc2_ref, hs2_ref):
    dis = _dis_from(degp_ref)
    c1 = _leaky(dis * (p_ref[0] + p_ref[1] + hs1_ref[...]))
    nrm = jnp.sqrt(jnp.sum(c1 * c1, axis=1, keepdims=True))
    hn = c1 / nrm
    ht2 = jnp.dot(hn, wc2_ref[...],
                  preferred_element_type=jnp.float32) + bc2_ref[...]
    hs2_ref[...] = ht2 * dis


def _tc_final_body(q_ref, hs2_ref, degp_ref, out_ref):
    dis = _dis_from(degp_ref)
    out_ref[...] = _leaky(dis * (q_ref[0] + q_ref[1] + hs2_ref[...]))


def _full(shape):
    return pl.BlockSpec(shape, lambda i: (0,) * len(shape))


_row_spec = lambda w: pl.BlockSpec((BLK, w), lambda i: (i, 0))
_part_spec = lambda w: pl.BlockSpec((2, BLK, w), lambda i: (0, i, 0))

_tc_mlp = pl.pallas_call(
    _tc_mlp_body,
    grid=(NBLK,),
    in_specs=[
        _row_spec(D_IN), _full((D_IN, H1)), _full((1, H1)),
        _full((H1, H2)), _full((1, H2)), _full((H2, C)), _full((1, C)),
        _part_spec(DEG_W),
    ],
    out_specs=_row_spec(C),
    out_shape=jax.ShapeDtypeStruct((N, C), jnp.float32),
)

_tc_mid = pl.pallas_call(
    _tc_mid_body,
    grid=(NBLK,),
    in_specs=[
        _part_spec(C), _row_spec(C), _part_spec(DEG_W),
        _full((C, C)), _full((1, C)),
    ],
    out_specs=_row_spec(C),
    out_shape=jax.ShapeDtypeStruct((N, C), jnp.float32),
)

_tc_final = pl.pallas_call(
    _tc_final_body,
    grid=(NBLK,),
    in_specs=[_part_spec(C), _row_spec(C), _part_spec(DEG_W)],
    out_specs=_row_spec(C),
    out_shape=jax.ShapeDtypeStruct((N, C), jnp.float32),
)


def kernel(x, edge_index, previous_embeddings_0, previous_embeddings_1,
           W1, b1, W2, b2, Wc1, bc1, Wc2, bc2):
    del previous_embeddings_0, previous_embeddings_1
    src = edge_index[0]
    dst = edge_index[1]
    pad_src = (jnp.arange(EPAD - E, dtype=jnp.int32) * 7) % N
    src_p = jnp.concatenate([src, pad_src]).reshape(NCHUNKS, CHUNK)
    dst_p = jnp.concatenate(
        [dst, jnp.full((EPAD - E,), N, jnp.int32)]).reshape(NCHUNKS, CHUNK)
    ones_deg = jnp.ones((CHUNK, DEG_W), jnp.float32)
    zeros_c = jnp.zeros((NPAD, C), jnp.float32)

    degp = _sc_degree(dst_p, ones_deg, zeros_c)
    hs1 = _tc_mlp(x, W1, b1.reshape(1, -1), W2, b2.reshape(1, -1),
                  Wc1, bc1.reshape(1, -1), degp)
    p1 = _sc_edge_scatter(hs1, src_p, dst_p, zeros_c)
    hs2 = _tc_mid(p1, hs1, degp, Wc2, bc2.reshape(1, -1))
    p2 = _sc_edge_scatter(hs2, src_p, dst_p, zeros_c)
    return _tc_final(p2, hs2, degp)


# speedup vs baseline: 2.9325x; 1.0327x over previous
import functools

import jax
import jax.numpy as jnp
from jax import lax
from jax.experimental import pallas as pl
from jax.experimental.pallas import tpu as pltpu
from jax.experimental.pallas import tpu_sc as plsc

N = 10000
D_IN = 256
H1 = 256
H2 = 256
C = 128

E = 320000
CHUNK = 128
NTILES = 32
CH_PER_TILE = 8 * (-(-E // (CHUNK * NTILES * 8)))
NCHUNKS = CH_PER_TILE * NTILES
EPAD = NCHUNKS * CHUNK
PIECE = 40
NP_FAST = 2
NP_SLOW = 2
FAST_CHUNKS = 16 * NP_FAST * PIECE
assert 16 * (NP_FAST + NP_SLOW) * PIECE == NCHUNKS
NPAD = 10112
ROWS_PER_TILE = NPAD // 16
DEG_W = C

BLK = 1000
NBLK = N // BLK

_mesh = plsc.VectorSubcoreMesh(core_axis_name="c", subcore_axis_name="s")


def _splat(buf, value):
    vec = jnp.full((16,), value, jnp.float32)

    def row(i, carry):
        for k in range(8):
            buf[i, pl.ds(k * 16, 16)] = vec
        return carry

    lax.fori_loop(0, CHUNK, row, 0)


def _init_acc(buf, acc, rbase):
    nfull = ROWS_PER_TILE // CHUNK
    rem = ROWS_PER_TILE - nfull * CHUNK
    for k in range(nfull):
        pltpu.sync_copy(buf, acc.at[pl.ds(rbase + k * CHUNK, CHUNK)])
    pltpu.sync_copy(buf.at[pl.ds(0, rem)],
                    acc.at[pl.ds(rbase + nfull * CHUNK, rem)])


@functools.partial(
    pl.kernel,
    mesh=_mesh,
    out_type=jax.ShapeDtypeStruct((2, NPAD, DEG_W), jnp.float32),
    scratch_types=[
        pltpu.VMEM((CH_PER_TILE, CHUNK), jnp.int32),
        pltpu.VMEM((CHUNK, DEG_W), jnp.float32),
        pltpu.VMEM((CHUNK, DEG_W), jnp.float32),
    ]
    + [pltpu.VMEM_SHARED((NPAD, DEG_W), jnp.float32)],
)
def _sc_degree(dst_hbm, out_hbm, dst_buf, ones_buf, zbuf, acc):
    c = lax.axis_index("c")
    s = lax.axis_index("s")
    rbase = s * ROWS_PER_TILE
    _splat(zbuf, 0.0)
    _splat(ones_buf, 1.0)
    _init_acc(zbuf, acc, rbase)
    cbase = (c * 16 + s) * CH_PER_TILE
    pltpu.sync_copy(dst_hbm.at[pl.ds(cbase, CH_PER_TILE)], dst_buf)
    plsc.subcore_barrier()

    def body(j, carry):
        pltpu.sync_copy(ones_buf, acc.at[dst_buf.at[j]], add=True)
        return carry

    lax.fori_loop(0, CH_PER_TILE, body, 0)
    plsc.subcore_barrier()
    pltpu.sync_copy(acc.at[pl.ds(rbase, ROWS_PER_TILE)],
                    out_hbm.at[c].at[pl.ds(rbase, ROWS_PER_TILE)])


@functools.partial(
    pl.kernel,
    mesh=_mesh,
    out_type=jax.ShapeDtypeStruct((2, NPAD, C), jnp.float32),
    scratch_types=[
        pltpu.VMEM((PIECE, CHUNK), jnp.int32),
        pltpu.VMEM((PIECE, CHUNK), jnp.int32),
    ]
    + [pltpu.VMEM((CHUNK, C), jnp.float32) for _ in range(2)]
    + [pltpu.VMEM_SHARED((NPAD, C), jnp.float32)]
    + [pltpu.SemaphoreType.DMA for _ in range(2)],
)
def _sc_edge_scatter(hs_hbm, src_hbm, dst_hbm, out_hbm,
                     src_buf, dst_buf, rb0, rb1, acc, g0, g1):
    NBUF = 2
    NGRP = PIECE // NBUF
    rbufs = (rb0, rb1)
    gsems = (g0, g1)
    c = lax.axis_index("c")
    s = lax.axis_index("s")
    rbase = s * ROWS_PER_TILE
    _splat(rb0, 0.0)
    _init_acc(rb0, acc, rbase)
    plsc.subcore_barrier()
    npieces = jnp.where(c == 0, NP_FAST, NP_SLOW)
    tbase = jnp.where(c == 0, s * NP_FAST * PIECE,
                      FAST_CHUNKS + s * NP_SLOW * PIECE)

    def piece(p, carry):
        pbase = tbase + p * PIECE
        pltpu.sync_copy(src_hbm.at[pl.ds(pbase, PIECE)], src_buf)
        pltpu.sync_copy(dst_hbm.at[pl.ds(pbase, PIECE)], dst_buf)
        for b in range(NBUF):
            pltpu.async_copy(hs_hbm.at[src_buf.at[b]], rbufs[b], gsems[b])

        def group(g, carry2):
            for b in range(NBUF):
                j = g * NBUF + b
                pltpu.make_async_copy(hs_hbm.at[src_buf.at[j]],
                                      rbufs[b], gsems[b]).wait()
                pltpu.sync_copy(rbufs[b], acc.at[dst_buf.at[j]], add=True)
                pltpu.async_copy(
                    hs_hbm.at[src_buf.at[lax.rem(j + NBUF, PIECE)]],
                    rbufs[b], gsems[b])
            return carry2

        lax.fori_loop(0, NGRP, group, 0)
        for b in range(NBUF):
            pltpu.make_async_copy(hs_hbm.at[src_buf.at[b]],
                                  rbufs[b], gsems[b]).wait()
        return carry

    lax.fori_loop(0, npieces, piece, 0)
    plsc.subcore_barrier()
    pltpu.sync_copy(acc.at[pl.ds(rbase, ROWS_PER_TILE)],
                    out_hbm.at[c].at[pl.ds(rbase, ROWS_PER_TILE)])


def _dis_from(degp_ref):
    deg = degp_ref[0, :, :1] + degp_ref[1, :, :1] + 1.0
    return lax.rsqrt(deg)


def _leaky(v):
    return jnp.where(v > 0, v, 0.01 * v)


def _tc_mlp_body(x_ref, w1_ref, b1_ref, w2_ref, b2_ref, wc1_ref, bc1_ref,
                 degp_ref, hs_ref):
    h = _leaky(jnp.dot(x_ref[...], w1_ref[...],
                       preferred_element_type=jnp.float32) + b1_ref[...])
    h = _leaky(jnp.dot(h, w2_ref[...],
                       preferred_element_type=jnp.float32) + b2_ref[...])
    ht = jnp.dot(h, wc1_ref[...],
                 preferred_element_type=jnp.float32) + bc1_ref[...]
    hs_ref[...] = ht * _dis_from(degp_ref)


def _tc_mid_body(p_ref, hs1_ref, degp_ref, wc2_ref, bc2_ref, hs2_ref):
    dis = _dis_from(degp_ref)
    c1 = _leaky(dis * (p_ref[0] + p_ref[1] + hs1_ref[...]))
    nrm = jnp.sqrt(jnp.sum(c1 * c1, axis=1, keepdims=True))
    hn = c1 / nrm
    ht2 = jnp.dot(hn, wc2_ref[...],
                  preferred_element_type=jnp.float32) + bc2_ref[...]
    hs2_ref[...] = ht2 * dis


def _tc_final_body(q_ref, hs2_ref, degp_ref, out_ref):
    dis = _dis_from(degp_ref)
    out_ref[...] = _leaky(dis * (q_ref[0] + q_ref[1] + hs2_ref[...]))


def _full(shape):
    return pl.BlockSpec(shape, lambda i: (0,) * len(shape))


_row_spec = lambda w: pl.BlockSpec((BLK, w), lambda i: (i, 0))
_part_spec = lambda w: pl.BlockSpec((2, BLK, w), lambda i: (0, i, 0))

_tc_mlp = pl.pallas_call(
    _tc_mlp_body,
    grid=(NBLK,),
    in_specs=[
        _row_spec(D_IN), _full((D_IN, H1)), _full((1, H1)),
        _full((H1, H2)), _full((1, H2)), _full((H2, C)), _full((1, C)),
        _part_spec(DEG_W),
    ],
    out_specs=_row_spec(C),
    out_shape=jax.ShapeDtypeStruct((N, C), jnp.float32),
)

_tc_mid = pl.pallas_call(
    _tc_mid_body,
    grid=(NBLK,),
    in_specs=[
        _part_spec(C), _row_spec(C), _part_spec(DEG_W),
        _full((C, C)), _full((1, C)),
    ],
    out_specs=_row_spec(C),
    out_shape=jax.ShapeDtypeStruct((N, C), jnp.float32),
)

_tc_final = pl.pallas_call(
    _tc_final_body,
    grid=(NBLK,),
    in_specs=[_part_spec(C), _row_spec(C), _part_spec(DEG_W)],
    out_specs=_row_spec(C),
    out_shape=jax.ShapeDtypeStruct((N, C), jnp.float32),
)


def kernel(x, edge_index, previous_embeddings_0, previous_embeddings_1,
           W1, b1, W2, b2, Wc1, bc1, Wc2, bc2):
    del previous_embeddings_0, previous_embeddings_1
    src = edge_index[0]
    dst = edge_index[1]
    pad_src = (jnp.arange(EPAD - E, dtype=jnp.int32) * 7) % N
    src_p = jnp.concatenate([src, pad_src]).reshape(NCHUNKS, CHUNK)
    dst_p = jnp.concatenate(
        [dst, jnp.full((EPAD - E,), N, jnp.int32)]).reshape(NCHUNKS, CHUNK)

    degp = _sc_degree(dst_p)
    hs1 = _tc_mlp(x, W1, b1.reshape(1, -1), W2, b2.reshape(1, -1),
                  Wc1, bc1.reshape(1, -1), degp)
    p1 = _sc_edge_scatter(hs1, src_p, dst_p)
    hs2 = _tc_mid(p1, hs1, degp, Wc2, bc2.reshape(1, -1))
    p2 = _sc_edge_scatter(hs2, src_p, dst_p)
    return _tc_final(p2, hs2, degp)


# speedup vs baseline: 2.9399x; 1.0025x over previous
import functools

import jax
import jax.numpy as jnp
from jax import lax
from jax.experimental import pallas as pl
from jax.experimental.pallas import tpu as pltpu
from jax.experimental.pallas import tpu_sc as plsc

N = 10000
D_IN = 256
H1 = 256
H2 = 256
C = 128

E = 320000
CHUNK = 128
NTILES = 32
CH_PER_TILE = 8 * (-(-E // (CHUNK * NTILES * 8)))
NCHUNKS = CH_PER_TILE * NTILES
EPAD = NCHUNKS * CHUNK
PIECE = 40
NP_FAST = 2
NP_SLOW = 2
FAST_CHUNKS = 16 * NP_FAST * PIECE
assert 16 * (NP_FAST + NP_SLOW) * PIECE == NCHUNKS
NPAD = 10112
ROWS_PER_TILE = NPAD // 16
DEG_W = C

BLK = 1000
NBLK = N // BLK

_mesh = plsc.VectorSubcoreMesh(core_axis_name="c", subcore_axis_name="s")


def _splat(buf, value):
    vec = jnp.full((16,), value, jnp.float32)

    def row(i, carry):
        for k in range(8):
            buf[i, pl.ds(k * 16, 16)] = vec
        return carry

    lax.fori_loop(0, CHUNK, row, 0)


def _init_acc(buf, acc, rbase):
    nfull = ROWS_PER_TILE // CHUNK
    rem = ROWS_PER_TILE - nfull * CHUNK
    for k in range(nfull):
        pltpu.sync_copy(buf, acc.at[pl.ds(rbase + k * CHUNK, CHUNK)])
    pltpu.sync_copy(buf.at[pl.ds(0, rem)],
                    acc.at[pl.ds(rbase + nfull * CHUNK, rem)])


@functools.partial(
    pl.kernel,
    mesh=_mesh,
    out_type=jax.ShapeDtypeStruct((2, NPAD, DEG_W), jnp.float32),
    scratch_types=[
        pltpu.VMEM((CH_PER_TILE, CHUNK), jnp.int32),
        pltpu.VMEM((CHUNK, DEG_W), jnp.float32),
        pltpu.VMEM((CHUNK, DEG_W), jnp.float32),
    ]
    + [pltpu.VMEM_SHARED((NPAD, DEG_W), jnp.float32)],
)
def _sc_degree(dst_hbm, out_hbm, dst_buf, ones_buf, zbuf, acc):
    c = lax.axis_index("c")
    s = lax.axis_index("s")
    rbase = s * ROWS_PER_TILE
    _splat(zbuf, 0.0)
    _splat(ones_buf, 1.0)
    _init_acc(zbuf, acc, rbase)
    cbase = (c * 16 + s) * CH_PER_TILE
    pltpu.sync_copy(dst_hbm.at[pl.ds(cbase, CH_PER_TILE)], dst_buf)
    plsc.subcore_barrier()

    def body(j, carry):
        pltpu.sync_copy(ones_buf, acc.at[dst_buf.at[j]], add=True)
        return carry

    lax.fori_loop(0, CH_PER_TILE, body, 0)
    plsc.subcore_barrier()
    pltpu.sync_copy(acc.at[pl.ds(rbase, ROWS_PER_TILE)],
                    out_hbm.at[c].at[pl.ds(rbase, ROWS_PER_TILE)])


@functools.partial(
    pl.kernel,
    mesh=_mesh,
    out_type=jax.ShapeDtypeStruct((2, NPAD, C), jnp.float32),
    scratch_types=[
        pltpu.VMEM((PIECE, CHUNK), jnp.int32),
        pltpu.VMEM((PIECE, CHUNK), jnp.int32),
    ]
    + [pltpu.VMEM((CHUNK, C), jnp.float32) for _ in range(2)]
    + [pltpu.VMEM_SHARED((NPAD, C), jnp.float32)]
    + [pltpu.SemaphoreType.DMA for _ in range(2)],
)
def _sc_edge_scatter(hs_hbm, src_hbm, dst_hbm, out_hbm,
                     src_buf, dst_buf, rb0, rb1, acc, g0, g1):
    NBUF = 2
    NGRP = PIECE // NBUF
    rbufs = (rb0, rb1)
    gsems = (g0, g1)
    c = lax.axis_index("c")
    s = lax.axis_index("s")
    rbase = s * ROWS_PER_TILE
    _splat(rb0, 0.0)
    _init_acc(rb0, acc, rbase)
    plsc.subcore_barrier()
    npieces = jnp.where(c == 0, NP_FAST, NP_SLOW)
    tbase = jnp.where(c == 0, s * NP_FAST * PIECE,
                      FAST_CHUNKS + s * NP_SLOW * PIECE)

    def piece(p, carry):
        pbase = tbase + p * PIECE
        pltpu.sync_copy(src_hbm.at[pl.ds(pbase, PIECE)], src_buf)
        pltpu.sync_copy(dst_hbm.at[pl.ds(pbase, PIECE)], dst_buf)
        for b in range(NBUF):
            pltpu.async_copy(hs_hbm.at[src_buf.at[b]], rbufs[b], gsems[b])

        def group(g, carry2):
            for b in range(NBUF):
                j = g * NBUF + b
                pltpu.make_async_copy(hs_hbm.at[src_buf.at[j]],
                                      rbufs[b], gsems[b]).wait()
                pltpu.sync_copy(rbufs[b], acc.at[dst_buf.at[j]], add=True)
                pltpu.async_copy(
                    hs_hbm.at[src_buf.at[lax.rem(j + NBUF, PIECE)]],
                    rbufs[b], gsems[b])
            return carry2

        lax.fori_loop(0, NGRP, group, 0)
        for b in range(NBUF):
            pltpu.make_async_copy(hs_hbm.at[src_buf.at[b]],
                                  rbufs[b], gsems[b]).wait()
        return carry

    lax.fori_loop(0, npieces, piece, 0)
    plsc.subcore_barrier()
    pltpu.sync_copy(acc.at[pl.ds(rbase, ROWS_PER_TILE)],
                    out_hbm.at[c].at[pl.ds(rbase, ROWS_PER_TILE)])


def _dis_from(degp_ref):
    deg = degp_ref[0, :, :1] + degp_ref[1, :, :1] + 1.0
    return lax.rsqrt(deg)


def _leaky(v):
    return jnp.where(v > 0, v, 0.01 * v)


def _tc_mlp_pre_body(x_ref, w1_ref, b1_ref, w2_ref, b2_ref, h2_ref):
    h = _leaky(jnp.dot(x_ref[...], w1_ref[...],
                       preferred_element_type=jnp.float32) + b1_ref[...])
    h2_ref[...] = _leaky(jnp.dot(h, w2_ref[...],
                                 preferred_element_type=jnp.float32)
                         + b2_ref[...])


def _tc_mlp_post_body(h2_ref, wc1_ref, bc1_ref, degp_ref, hs_ref):
    ht = jnp.dot(h2_ref[...], wc1_ref[...],
                 preferred_element_type=jnp.float32) + bc1_ref[...]
    hs_ref[...] = ht * _dis_from(degp_ref)


def _tc_mid_body(p_ref, hs1_ref, degp_ref, wc2_ref, bc2_ref, hs2_ref):
    dis = _dis_from(degp_ref)
    c1 = _leaky(dis * (p_ref[0] + p_ref[1] + hs1_ref[...]))
    nrm = jnp.sqrt(jnp.sum(c1 * c1, axis=1, keepdims=True))
    hn = c1 / nrm
    ht2 = jnp.dot(hn, wc2_ref[...],
                  preferred_element_type=jnp.float32) + bc2_ref[...]
    hs2_ref[...] = ht2 * dis


def _tc_final_body(q_ref, hs2_ref, degp_ref, out_ref):
    dis = _dis_from(degp_ref)
    out_ref[...] = _leaky(dis * (q_ref[0] + q_ref[1] + hs2_ref[...]))


def _full(shape):
    return pl.BlockSpec(shape, lambda i: (0,) * len(shape))


_row_spec = lambda w: pl.BlockSpec((BLK, w), lambda i: (i, 0))
_part_spec = lambda w: pl.BlockSpec((2, BLK, w), lambda i: (0, i, 0))

_tc_mlp_pre = pl.pallas_call(
    _tc_mlp_pre_body,
    grid=(NBLK,),
    in_specs=[
        _row_spec(D_IN), _full((D_IN, H1)), _full((1, H1)),
        _full((H1, H2)), _full((1, H2)),
    ],
    out_specs=_row_spec(H2),
    out_shape=jax.ShapeDtypeStruct((N, H2), jnp.float32),
)

_tc_mlp_post = pl.pallas_call(
    _tc_mlp_post_body,
    grid=(NBLK,),
    in_specs=[
        _row_spec(H2), _full((H2, C)), _full((1, C)), _part_spec(DEG_W),
    ],
    out_specs=_row_spec(C),
    out_shape=jax.ShapeDtypeStruct((N, C), jnp.float32),
)

_tc_mid = pl.pallas_call(
    _tc_mid_body,
    grid=(NBLK,),
    in_specs=[
        _part_spec(C), _row_spec(C), _part_spec(DEG_W),
        _full((C, C)), _full((1, C)),
    ],
    out_specs=_row_spec(C),
    out_shape=jax.ShapeDtypeStruct((N, C), jnp.float32),
)

_tc_final = pl.pallas_call(
    _tc_final_body,
    grid=(NBLK,),
    in_specs=[_part_spec(C), _row_spec(C), _part_spec(DEG_W)],
    out_specs=_row_spec(C),
    out_shape=jax.ShapeDtypeStruct((N, C), jnp.float32),
)


def kernel(x, edge_index, previous_embeddings_0, previous_embeddings_1,
           W1, b1, W2, b2, Wc1, bc1, Wc2, bc2):
    del previous_embeddings_0, previous_embeddings_1
    src = edge_index[0]
    dst = edge_index[1]
    pad_src = (jnp.arange(EPAD - E, dtype=jnp.int32) * 7) % N
    src_p = jnp.concatenate([src, pad_src]).reshape(NCHUNKS, CHUNK)
    dst_p = jnp.concatenate(
        [dst, jnp.full((EPAD - E,), N, jnp.int32)]).reshape(NCHUNKS, CHUNK)

    degp = _sc_degree(dst_p)
    h2 = _tc_mlp_pre(x, W1, b1.reshape(1, -1), W2, b2.reshape(1, -1))
    hs1 = _tc_mlp_post(h2, Wc1, bc1.reshape(1, -1), degp)
    p1 = _sc_edge_scatter(hs1, src_p, dst_p)
    hs2 = _tc_mid(p1, hs1, degp, Wc2, bc2.reshape(1, -1))
    p2 = _sc_edge_scatter(hs2, src_p, dst_p)
    return _tc_final(p2, hs2, degp)
